# Initial kernel scaffold; baseline (speedup 1.0000x reference)
import jax
import jax.numpy as jnp
from jax.experimental import pallas as pl


def kernel(xyz, feat_in, W0, b0, g0, beta0, W1, b1, g1, beta1, W2, b2, g2, beta2):
    raise NotImplementedError("write your pallas kernel here")


# speedup vs baseline: 7.8038x; 7.8038x over previous
import functools

import jax
import jax.numpy as jnp
from jax import lax
from jax.experimental import pallas as pl
from jax.experimental.pallas import tpu as pltpu
from jax.experimental.pallas import tpu_sc as plsc

_B, _N, _CIN = 4, 8192, 16
_M = _N // 4
_K = 32
_MBLK = 128
_RBLK = 2048
_D = 32
_ROWS = _B * _M * _K
_NW = 32
_RPW = _ROWS // _NW
_BIG = 3.0e38
_EPS = 1e-5


def _knn_kernel(cent_ref, xyzt_ref, idx_ref, cent_out_ref, s_scr):
    b = pl.program_id(0)
    c = cent_ref[0]
    p = xyzt_ref[0]
    pn2 = jnp.sum(p * p, axis=0)
    s = pn2[None, :] - 2.0 * jnp.dot(c, p, preferred_element_type=jnp.float32)
    s_scr[...] = s
    iota = lax.broadcasted_iota(jnp.int32, (_MBLK, _N), 1)
    kiota = lax.broadcasted_iota(jnp.int32, (_MBLK, _K), 1)
    ibig = jnp.int32(2 ** 30)

    def body(j, acc):
        s_cur = s_scr[...]
        rowmin = jnp.min(s_cur, axis=1)
        t = jnp.where(s_cur == rowmin[:, None], iota, ibig)
        win = jnp.min(t, axis=1)
        s_scr[...] = jnp.where(t == win[:, None], _BIG, s_cur)
        return jnp.where(kiota == j, win[:, None], acc)

    acc = lax.fori_loop(0, _K, body, jnp.zeros((_MBLK, _K), jnp.int32))
    idx_ref[0] = acc + b * _N
    cent_out_ref[0] = c


def _knn(cent, xyzt):
    grid = (_B, _M // _MBLK)
    return pl.pallas_call(
        _knn_kernel,
        grid=grid,
        in_specs=[
            pl.BlockSpec((1, _MBLK, 3), lambda b, i: (b, i, 0)),
            pl.BlockSpec((1, 3, _N), lambda b, i: (b, 0, 0)),
        ],
        out_specs=[
            pl.BlockSpec((1, _MBLK, _K), lambda b, i: (b, i, 0)),
            pl.BlockSpec((1, _MBLK, 3), lambda b, i: (b, i, 0)),
        ],
        out_shape=[
            jax.ShapeDtypeStruct((_B, _M, _K), jnp.int32),
            jax.ShapeDtypeStruct((_B, _M, 3), jnp.float32),
        ],
        scratch_shapes=[pltpu.VMEM((_MBLK, _N), jnp.float32)],
    )(cent, xyzt)


def _gather(table, idx3d):
    mesh = plsc.VectorSubcoreMesh(core_axis_name="c", subcore_axis_name="s")

    @functools.partial(
        pl.kernel,
        mesh=mesh,
        compiler_params=pltpu.CompilerParams(use_tc_tiling_on_sc=False),
        out_type=jax.ShapeDtypeStruct((_ROWS, _D), jnp.float32),
        scratch_types=[
            pltpu.VMEM((_RPW // 128, 128), jnp.int32),
            pltpu.VMEM((1024, _D), jnp.float32),
            pltpu.SemaphoreType.DMA,
        ],
    )
    def gk(idx_hbm, table_hbm, out_hbm, idx_v, rows_v, sem):
        wid = lax.axis_index("s") * 2 + lax.axis_index("c")
        base = wid * _RPW
        pltpu.sync_copy(idx_hbm.at[wid], idx_v)

        def outer(o, carry):
            cps = []
            for j in range(8):
                cps.append(pltpu.async_copy(
                    table_hbm.at[idx_v.at[o * 8 + j]],
                    rows_v.at[pl.ds(j * 128, 128)], sem))
            for cp in cps:
                cp.wait()
            pltpu.sync_copy(rows_v, out_hbm.at[pl.ds(base + o * 1024, 1024)])
            return carry

        lax.fori_loop(0, _RPW // 1024, outer, 0)

    return gk(idx3d, table)


def _acc_stats(st_ref, pre):
    s1 = jnp.sum(pre, axis=0)
    s2 = jnp.sum(pre * pre, axis=0)
    blk = jnp.concatenate([s1[None, :], s2[None, :]], axis=0)

    @pl.when(pl.program_id(0) == 0)
    def _():
        st_ref[...] = blk

    @pl.when(pl.program_id(0) != 0)
    def _():
        st_ref[...] = st_ref[...] + blk


def _bn_in(st, gamma, beta):
    mu = st[0:1, :] * (1.0 / _ROWS)
    var = st[1:2, :] * (1.0 / _ROWS) - mu * mu
    inv = lax.rsqrt(var + _EPS) * gamma
    return mu, inv, beta


def _l0_kernel(g_ref, cent_ref, w0pt_ref, w0xt_ref, b0_ref, pre_ref, st_ref):
    c = cent_ref[...]
    corr = jnp.dot(c, w0xt_ref[...], preferred_element_type=jnp.float32)
    corr = jnp.broadcast_to(corr[:, None, :], (_RBLK // _K, _K, 32))
    corr = corr.reshape(_RBLK, 32)
    pre = (jnp.dot(g_ref[...], w0pt_ref[...],
                   preferred_element_type=jnp.float32)
           - corr + b0_ref[...])
    pre_ref[...] = pre
    _acc_stats(st_ref, pre)


def _l0(g, cent_flat, w0pt, w0xt, b0):
    nblk = _ROWS // _RBLK
    return pl.pallas_call(
        _l0_kernel,
        grid=(nblk,),
        in_specs=[
            pl.BlockSpec((_RBLK, _D), lambda i: (i, 0)),
            pl.BlockSpec((_RBLK // _K, 3), lambda i: (i, 0)),
            pl.BlockSpec((_D, 32), lambda i: (0, 0)),
            pl.BlockSpec((3, 32), lambda i: (0, 0)),
            pl.BlockSpec((1, 32), lambda i: (0, 0)),
        ],
        out_specs=[
            pl.BlockSpec((_RBLK, 32), lambda i: (i, 0)),
            pl.BlockSpec((2, 32), lambda i: (0, 0)),
        ],
        out_shape=[
            jax.ShapeDtypeStruct((_ROWS, 32), jnp.float32),
            jax.ShapeDtypeStruct((2, 32), jnp.float32),
        ],
    )(g, cent_flat, w0pt, w0xt, b0)


def _mid_kernel(pre_in_ref, st_in_ref, g_ref, be_ref, wt_ref, b_ref,
                pre_ref, st_ref):
    mu, inv, beta = _bn_in(st_in_ref[...], g_ref[...], be_ref[...])
    z = jnp.maximum((pre_in_ref[...] - mu) * inv + beta, 0.0)
    pre = jnp.dot(z, wt_ref[...], preferred_element_type=jnp.float32) + b_ref[...]
    pre_ref[...] = pre
    _acc_stats(st_ref, pre)


def _mid(pre_in, st_in, gamma, beta, wt, b, cout):
    nblk = _ROWS // _RBLK
    cin = pre_in.shape[1]
    return pl.pallas_call(
        _mid_kernel,
        grid=(nblk,),
        in_specs=[
            pl.BlockSpec((_RBLK, cin), lambda i: (i, 0)),
            pl.BlockSpec((2, cin), lambda i: (0, 0)),
            pl.BlockSpec((1, cin), lambda i: (0, 0)),
            pl.BlockSpec((1, cin), lambda i: (0, 0)),
            pl.BlockSpec((cin, cout), lambda i: (0, 0)),
            pl.BlockSpec((1, cout), lambda i: (0, 0)),
        ],
        out_specs=[
            pl.BlockSpec((_RBLK, cout), lambda i: (i, 0)),
            pl.BlockSpec((2, cout), lambda i: (0, 0)),
        ],
        out_shape=[
            jax.ShapeDtypeStruct((_ROWS, cout), jnp.float32),
            jax.ShapeDtypeStruct((2, cout), jnp.float32),
        ],
    )(pre_in, st_in, gamma, beta, wt, b)


def _fin_kernel(pre_in_ref, st_in_ref, g_ref, be_ref, out_ref):
    mu, inv, beta = _bn_in(st_in_ref[...], g_ref[...], be_ref[...])
    z = jnp.maximum((pre_in_ref[...] - mu) * inv + beta, 0.0)
    z3 = z.reshape(_RBLK // _K, _K, 64)
    out_ref[...] = jnp.max(z3, axis=1)


def _fin(pre_in, st_in, gamma, beta):
    nblk = _ROWS // _RBLK
    return pl.pallas_call(
        _fin_kernel,
        grid=(nblk,),
        in_specs=[
            pl.BlockSpec((_RBLK, 64), lambda i: (i, 0)),
            pl.BlockSpec((2, 64), lambda i: (0, 0)),
            pl.BlockSpec((1, 64), lambda i: (0, 0)),
            pl.BlockSpec((1, 64), lambda i: (0, 0)),
        ],
        out_specs=pl.BlockSpec((_RBLK // _K, 64), lambda i: (i, 0)),
        out_shape=jax.ShapeDtypeStruct((_B * _M, 64), jnp.float32),
    )(pre_in, st_in, gamma, beta)


def kernel(xyz, feat_in, W0, b0, g0, beta0, W1, b1, g1, beta1,
           W2, b2, g2, beta2):
    idx_center = jnp.linspace(0.0, _N - 1, _M).astype(jnp.int32)
    cent = jnp.take(xyz, idx_center, axis=1)
    xyzt = jnp.transpose(xyz, (0, 2, 1))

    knn_idx, centers_out = _knn(cent, xyzt)

    feat_t = jnp.transpose(feat_in, (0, 2, 1))
    table = jnp.concatenate(
        [xyz, feat_t, jnp.zeros((_B, _N, _D - 3 - _CIN), jnp.float32)],
        axis=-1).reshape(_B * _N, _D)
    idx3d = knn_idx.reshape(_NW, _RPW // 128, 128)

    g = _gather(table, idx3d)

    w0pt = jnp.concatenate(
        [W0.T, jnp.zeros((_D - 19, 32), jnp.float32)], axis=0)
    w0xt = W0[:, 0:3].T
    cent_flat = cent.reshape(_B * _M, 3)

    pre0, st0 = _l0(g, cent_flat, w0pt, w0xt, b0.reshape(1, 32))
    pre1, st1 = _mid(pre0, st0, g0.reshape(1, 32), beta0.reshape(1, 32),
                     W1.T, b1.reshape(1, 32), 32)
    pre2, st2 = _mid(pre1, st1, g1.reshape(1, 32), beta1.reshape(1, 32),
                     W2.T, b2.reshape(1, 64), 64)
    fflat = _fin(pre2, st2, g2.reshape(1, 64), beta2.reshape(1, 64))

    f = jnp.transpose(fflat.reshape(_B, _M, 64), (0, 2, 1))
    return centers_out, f
